# baseline (speedup 1.0000x reference)
import functools

import jax
import jax.numpy as jnp
from jax import lax
from jax.experimental import pallas as pl
from jax.experimental.pallas import tpu as pltpu
from jax.experimental.pallas import tpu_sc as plsc

_B = 16384
_F = 26
_V = 100000
_D = 64
_H1 = 256
_H2 = 128

_NC = 2
_NS = 16
_NW = _NC * _NS
_RPW = _B // _NW
_CB = 16
_MB = 128
_NMACRO = _RPW // _MB
_NMICRO = _MB // _CB
_OB = 64
_LANES = 16
_DV = _D // _LANES


def _sc_bi_interaction(xt, tables):
    mesh = plsc.VectorSubcoreMesh(core_axis_name="c", subcore_axis_name="s")

    @functools.partial(
        pl.kernel,
        out_type=jax.ShapeDtypeStruct((_B, _D), jnp.float32),
        mesh=mesh,
        compiler_params=pltpu.CompilerParams(use_tc_tiling_on_sc=True),
        scratch_types=[
            pltpu.VMEM((_F, _MB), jnp.int32),
            pltpu.VMEM((2, _F * _CB, _D), jnp.float32),
            pltpu.VMEM((_OB, _D), jnp.float32),
            pltpu.SemaphoreType.DMA,
        ],
    )
    def body(xt_hbm, tab_hbm, bi_hbm, idx_v, rows_v, bi_v, sem):
        wid = lax.axis_index("s") * _NC + lax.axis_index("c")
        row0 = wid * _RPW

        def issue(bb, mm):
            def issue_field(f, carry):
                ivec = idx_v[f, pl.ds(mm * _CB, _LANES)]
                for k in range(_LANES):
                    v = pl.multiple_of(ivec[k], 8)
                    pltpu.async_copy(
                        tab_hbm.at[v],
                        rows_v.at[bb, f * _CB + k],
                        sem,
                    )
                return carry

            lax.fori_loop(0, _F, issue_field, 0)

        def macro_body(g, carry):
            m0 = row0 + g * _MB
            pltpu.sync_copy(xt_hbm.at[:, pl.ds(m0, _MB)], idx_v)
            issue(0, 0)

            def micro_body(m, carry1):
                bb = m & 1
                pltpu.make_async_copy(
                    tab_hbm.at[pl.ds(0, _F * _CB)], rows_v.at[bb], sem
                ).wait()

                @pl.when(m < _NMICRO - 1)
                def _():
                    issue(1 - bb, m + 1)

                for k in range(_LANES):
                    zeros = jnp.zeros((_LANES,), jnp.float32)

                    def f_body(f, accs, k=k):
                        base = f * _CB + k
                        vs = [
                            rows_v[bb, base, pl.ds(d * _LANES, _LANES)]
                            for d in range(_DV)
                        ]
                        s = tuple(accs[d] + vs[d] for d in range(_DV))
                        q = tuple(
                            accs[_DV + d] + vs[d] * vs[d] for d in range(_DV)
                        )
                        return s + q

                    accs = lax.fori_loop(
                        0, _F, f_body, (zeros,) * (2 * _DV), unroll=13
                    )
                    rr = (m % (_OB // _CB)) * _CB + k
                    for d in range(_DV):
                        bi_v[rr, pl.ds(d * _LANES, _LANES)] = 0.5 * (
                            accs[d] * accs[d] - accs[_DV + d]
                        )

                @pl.when(m % (_OB // _CB) == (_OB // _CB) - 1)
                def _():
                    ob = m0 + (m // (_OB // _CB)) * _OB
                    pltpu.sync_copy(bi_v, bi_hbm.at[pl.ds(ob, _OB)])

                return carry1

            lax.fori_loop(0, _NMICRO, micro_body, 0)
            return carry

        lax.fori_loop(0, _NMACRO, macro_body, 0)

    return body(xt, tables)


def _tc_head(bi, x, W1, b1, W2, b2, W3t, b3, Wlt, bl):
    BT = 2048
    grid = (_B // BT,)

    def body(bi_ref, x_ref, W1_ref, b1_ref, W2_ref, b2_ref, W3_ref, b3_ref,
             Wl_ref, bl_ref, out_ref):
        h = jnp.dot(bi_ref[...], W1_ref[...],
                    preferred_element_type=jnp.float32)
        h = jnp.maximum(h + b1_ref[...], 0.0)
        h = jnp.dot(h, W2_ref[...],
                    preferred_element_type=jnp.float32)
        h = jnp.maximum(h + b2_ref[...], 0.0)
        deep = jnp.sum(h * W3_ref[...], axis=1, keepdims=True) + b3_ref[...]
        xf = x_ref[...].astype(jnp.float32)
        xb = xf.astype(jnp.bfloat16).astype(jnp.float32)
        wb = Wl_ref[...].astype(jnp.bfloat16).astype(jnp.float32)
        lin = jnp.sum(xb * wb, axis=1, keepdims=True) + bl_ref[...]
        out_ref[...] = jax.nn.sigmoid(lin + deep)

    full = lambda shape: pl.BlockSpec(shape, lambda i: (0, 0))
    return pl.pallas_call(
        body,
        grid=grid,
        in_specs=[
            pl.BlockSpec((BT, _D), lambda i: (i, 0)),
            pl.BlockSpec((BT, _F), lambda i: (i, 0)),
            full((_D, _H1)),
            full((1, _H1)),
            full((_H1, _H2)),
            full((1, _H2)),
            full((1, _H2)),
            full((1, 1)),
            full((1, _F)),
            full((1, 1)),
        ],
        out_specs=pl.BlockSpec((BT, 1), lambda i: (i, 0)),
        out_shape=jax.ShapeDtypeStruct((_B, 1), jnp.float32),
    )(bi, x, W1, b1, W2, b2, W3t, b3, Wlt, bl)


def kernel(x, tables, Wl, bl, W1, b1, W2, b2, W3, b3):
    x = x.astype(jnp.int32)
    xt = jnp.transpose(x) + (jnp.arange(_F, dtype=jnp.int32) * _V)[:, None]

    bi = _sc_bi_interaction(xt, tables.reshape(_F * _V, _D))

    out = _tc_head(
        bi, x,
        W1, b1.reshape(1, _H1),
        W2, b2.reshape(1, _H2),
        W3.reshape(1, _H2), b3.reshape(1, 1),
        Wl.reshape(1, _F), bl.reshape(1, 1),
    )
    return out


# speedup vs baseline: 1.2058x; 1.2058x over previous
import functools

import jax
import jax.numpy as jnp
from jax import lax
from jax.experimental import pallas as pl
from jax.experimental.pallas import tpu as pltpu
from jax.experimental.pallas import tpu_sc as plsc

_B = 16384
_F = 26
_V = 100000
_D = 64
_H1 = 256
_H2 = 128

_NC = 2
_NS = 16
_NW = _NC * _NS
_RPW = _B // _NW
_CB = 16
_MB = 128
_NMACRO = _RPW // _MB
_NMICRO = _MB // _CB
_OB = 64
_LANES = 16
_DV = _D // _LANES


def _sc_bi_interaction(xt, tables):
    mesh = plsc.VectorSubcoreMesh(core_axis_name="c", subcore_axis_name="s")

    @functools.partial(
        pl.kernel,
        out_type=jax.ShapeDtypeStruct((_B, _D), jnp.float32),
        mesh=mesh,
        compiler_params=pltpu.CompilerParams(use_tc_tiling_on_sc=True),
        scratch_types=[
            pltpu.VMEM((_F, _MB), jnp.int32),
            pltpu.VMEM((2, _F * _CB, _D), jnp.float32),
            pltpu.VMEM((_OB, _D), jnp.float32),
            pltpu.SemaphoreType.DMA,
        ],
    )
    def body(xt_hbm, tab_hbm, bi_hbm, idx_v, rows_v, bi_v, sem):
        wid = lax.axis_index("s") * _NC + lax.axis_index("c")
        row0 = wid * _RPW

        def issue(bb, mm):
            def issue_field(f, carry):
                ivec = idx_v[f, pl.ds(mm * _CB, _LANES)]
                for k in range(_LANES):
                    v = pl.multiple_of(ivec[k], 8)
                    pltpu.async_copy(
                        tab_hbm.at[v],
                        rows_v.at[bb, f * _CB + k],
                        sem,
                    )
                return carry

            lax.fori_loop(0, _F, issue_field, 0)

        def macro_body(g, carry):
            m0 = row0 + g * _MB
            pltpu.sync_copy(xt_hbm.at[:, pl.ds(m0, _MB)], idx_v)
            issue(0, 0)

            def micro_body(m, carry1):
                bb = m & 1
                pltpu.make_async_copy(
                    tab_hbm.at[pl.ds(0, _F * _CB)], rows_v.at[bb], sem
                ).wait()

                @pl.when(m < _NMICRO - 1)
                def _():
                    issue(1 - bb, m + 1)

                for k in range(_LANES):
                    zeros = jnp.zeros((_LANES,), jnp.float32)

                    def f_body(f, accs, k=k):
                        base = f * _CB + k
                        vs = [
                            rows_v[bb, base, pl.ds(d * _LANES, _LANES)]
                            for d in range(_DV)
                        ]
                        s = tuple(accs[d] + vs[d] for d in range(_DV))
                        q = tuple(
                            accs[_DV + d] + vs[d] * vs[d] for d in range(_DV)
                        )
                        return s + q

                    accs = lax.fori_loop(
                        0, _F, f_body, (zeros,) * (2 * _DV), unroll=2
                    )
                    rr = (m % (_OB // _CB)) * _CB + k
                    for d in range(_DV):
                        bi_v[rr, pl.ds(d * _LANES, _LANES)] = 0.5 * (
                            accs[d] * accs[d] - accs[_DV + d]
                        )

                @pl.when(m % (_OB // _CB) == (_OB // _CB) - 1)
                def _():
                    ob = m0 + (m // (_OB // _CB)) * _OB
                    pltpu.sync_copy(bi_v, bi_hbm.at[pl.ds(ob, _OB)])

                return carry1

            lax.fori_loop(0, _NMICRO, micro_body, 0)
            return carry

        lax.fori_loop(0, _NMACRO, macro_body, 0)

    return body(xt, tables)


def _tc_head(bi, x, W1, b1, W2, b2, W3t, b3, Wlt, bl):
    BT = 2048
    grid = (_B // BT,)

    def body(bi_ref, x_ref, W1_ref, b1_ref, W2_ref, b2_ref, W3_ref, b3_ref,
             Wl_ref, bl_ref, out_ref):
        h = jnp.dot(bi_ref[...], W1_ref[...],
                    preferred_element_type=jnp.float32)
        h = jnp.maximum(h + b1_ref[...], 0.0)
        h = jnp.dot(h, W2_ref[...],
                    preferred_element_type=jnp.float32)
        h = jnp.maximum(h + b2_ref[...], 0.0)
        deep = jnp.sum(h * W3_ref[...], axis=1, keepdims=True) + b3_ref[...]
        xf = x_ref[...].astype(jnp.float32)
        xb = xf.astype(jnp.bfloat16).astype(jnp.float32)
        wb = Wl_ref[...].astype(jnp.bfloat16).astype(jnp.float32)
        lin = jnp.sum(xb * wb, axis=1, keepdims=True) + bl_ref[...]
        out_ref[...] = jax.nn.sigmoid(lin + deep)

    full = lambda shape: pl.BlockSpec(shape, lambda i: (0, 0))
    return pl.pallas_call(
        body,
        grid=grid,
        in_specs=[
            pl.BlockSpec((BT, _D), lambda i: (i, 0)),
            pl.BlockSpec((BT, _F), lambda i: (i, 0)),
            full((_D, _H1)),
            full((1, _H1)),
            full((_H1, _H2)),
            full((1, _H2)),
            full((1, _H2)),
            full((1, 1)),
            full((1, _F)),
            full((1, 1)),
        ],
        out_specs=pl.BlockSpec((BT, 1), lambda i: (i, 0)),
        out_shape=jax.ShapeDtypeStruct((_B, 1), jnp.float32),
    )(bi, x, W1, b1, W2, b2, W3t, b3, Wlt, bl)


def kernel(x, tables, Wl, bl, W1, b1, W2, b2, W3, b3):
    x = x.astype(jnp.int32)
    xt = jnp.transpose(x) + (jnp.arange(_F, dtype=jnp.int32) * _V)[:, None]

    bi = _sc_bi_interaction(xt, tables.reshape(_F * _V, _D))

    out = _tc_head(
        bi, x,
        W1, b1.reshape(1, _H1),
        W2, b2.reshape(1, _H2),
        W3.reshape(1, _H2), b3.reshape(1, 1),
        Wl.reshape(1, _F), bl.reshape(1, 1),
    )
    return out
